# baseline (speedup 1.0000x reference)
import dataclasses
import functools

import jax
import jax.numpy as jnp
from jax.experimental import pallas as pl
from jax.experimental.pallas import tpu as pltpu
from jax.experimental.pallas import tpu_sc as plsc

_B, _C, _H, _W = 16, 3, 512, 512
_HB = 512
_NJ = _H // _HB

_NTC = 8
_NSC = _B - _NTC
_NCORES, _NSUB, _L = 2, 16, 16
_NW = _NCORES * _NSUB
_WPB = _NW // _NSC
_RW = _H // _WPB
_RC = 16
_CHUNKS = _RW // _RC


def _tc_kernel(x_ref, acc_ref):
    j = pl.program_id(1)
    x = x_ref[0]
    c0, c1, c2 = x[0], x[1], x[2]

    d1 = c1 - c0
    d2 = c2 - c0
    r1 = jnp.exp(d1)
    r2 = jnp.exp(d2)
    inv = 1.0 / (1.0 + r1 + r2)
    p1sum = jnp.sum(r1 * inv)
    p2sum = jnp.sum(r2 * inv)

    a1 = (d1 > 0.0) & (d1 >= d2)
    a2 = (d2 > 0.0) & (d2 > d1)
    rowhas1 = jnp.any(a1, axis=1)
    rowhas2 = jnp.any(a2, axis=1)

    rows = (j * _HB + jax.lax.iota(jnp.int32, _HB)).astype(jnp.float32)
    big = jnp.float32(_H)
    ymin1 = jnp.min(jnp.where(rowhas1, rows, big))
    ymax1 = jnp.max(jnp.where(rowhas1, rows, -1.0))
    ymin2 = jnp.min(jnp.where(rowhas2, rows, big))
    ymax2 = jnp.max(jnp.where(rowhas2, rows, -1.0))

    lane = jax.lax.broadcasted_iota(jnp.int32, (1, 128), 1)
    vals = jnp.zeros((1, 128), jnp.float32)
    vals = jnp.where(lane == 0, ymin1, vals)
    vals = jnp.where(lane == 1, ymax1, vals)
    vals = jnp.where(lane == 2, ymin2, vals)
    vals = jnp.where(lane == 3, ymax2, vals)
    vals = jnp.where(lane == 4, p1sum, vals)
    vals = jnp.where(lane == 5, p2sum, vals)

    @pl.when(j == 0)
    def _():
        acc_ref[0] = vals

    @pl.when(j > 0)
    def _():
        prev = acc_ref[0]
        is_min = (lane == 0) | (lane == 2)
        is_max = (lane == 1) | (lane == 3)
        acc_ref[0] = jnp.where(is_min, jnp.minimum(prev, vals),
                               jnp.where(is_max, jnp.maximum(prev, vals),
                                         prev + vals))


def _sc_stats(x):
    mesh = plsc.VectorSubcoreMesh(core_axis_name="c", subcore_axis_name="s")
    cp = pltpu.CompilerParams()
    if "needs_layout_passes" in pltpu.CompilerParams.__dataclass_fields__:
        cp = dataclasses.replace(cp, needs_layout_passes=False)

    @functools.partial(
        pl.kernel, mesh=mesh, compiler_params=cp,
        out_type=jax.ShapeDtypeStruct((_NW, 16), jnp.float32),
        scratch_types=[
            pltpu.VMEM((2, _C, _RC, _W), jnp.float32),
            pltpu.VMEM((16,), jnp.float32),
            pltpu.SemaphoreType.DMA,
            pltpu.SemaphoreType.DMA,
        ],
    )
    def sc_kernel(x_hbm, out_hbm, buf, outv, sem0, sem1):
        wid = jax.lax.axis_index("c") * _NSUB + jax.lax.axis_index("s")
        b = _NTC + wid // _WPB
        row0 = (wid % _WPB) * _RW
        sems = (sem0, sem1)

        def issue(k):
            slot = k % 2
            rs = row0 + k * _RC
            return [pltpu.async_copy(x_hbm.at[b, c, pl.ds(rs, _RC), :],
                                     buf.at[slot, c], sems[slot])
                    for c in range(_C)]

        zero = jnp.zeros((16,), jnp.float32)
        bigv = jnp.full((16,), float(_H), jnp.float32)
        negv = jnp.full((16,), -1.0, jnp.float32)
        carry = (zero, zero, bigv, negv, bigv, negv)

        pending = issue(0)
        for k in range(_CHUNKS):
            nxt = issue(k + 1) if k + 1 < _CHUNKS else None
            for cp in pending:
                cp.wait()
            slot = k % 2
            rs = row0 + k * _RC

            def row_body(r, cr, slot=slot, rs=rs):
                rowv = jnp.full((16,), (rs + r).astype(jnp.float32))

                def w_body(wi, cr2):
                    p1, p2, y1n, y1x, y2n, y2x = cr2
                    w0 = wi * _L
                    v0 = buf[slot, 0, r, pl.ds(w0, _L)]
                    v1 = buf[slot, 1, r, pl.ds(w0, _L)]
                    v2 = buf[slot, 2, r, pl.ds(w0, _L)]
                    d1 = v1 - v0
                    d2 = v2 - v0
                    e1 = jnp.exp(d1)
                    e2 = jnp.exp(d2)
                    inv = 1.0 / (1.0 + e1 + e2)
                    p1 = p1 + e1 * inv
                    p2 = p2 + e2 * inv
                    m1 = (d1 > 0.0) & (d1 >= d2)
                    m2 = (d2 > 0.0) & (d2 > d1)
                    y1n = jnp.minimum(y1n, jnp.where(m1, rowv, bigv))
                    y1x = jnp.maximum(y1x, jnp.where(m1, rowv, negv))
                    y2n = jnp.minimum(y2n, jnp.where(m2, rowv, bigv))
                    y2x = jnp.maximum(y2x, jnp.where(m2, rowv, negv))
                    return (p1, p2, y1n, y1x, y2n, y2x)

                return jax.lax.fori_loop(0, _W // _L, w_body, cr)

            carry = jax.lax.fori_loop(0, _RC, row_body, carry)
            pending = nxt

        p1, p2, y1n, y1x, y2n, y2x = carry
        lane = jax.lax.iota(jnp.int32, 16)
        res = zero
        res = jnp.where(lane == 0, jnp.min(y1n), res)
        res = jnp.where(lane == 1, jnp.max(y1x), res)
        res = jnp.where(lane == 2, jnp.min(y2n), res)
        res = jnp.where(lane == 3, jnp.max(y2x), res)
        res = jnp.where(lane == 4, jnp.sum(p1), res)
        res = jnp.where(lane == 5, jnp.sum(p2), res)
        outv[...] = res
        pltpu.sync_copy(outv, out_hbm.at[wid])

    return sc_kernel(x)


@jax.jit
def kernel(segmentation_mask):
    x = segmentation_mask

    acc = pl.pallas_call(
        _tc_kernel,
        grid=(_NTC, _NJ),
        in_specs=[pl.BlockSpec((1, _C, _HB, _W), lambda b, j: (b, 0, j, 0))],
        out_specs=pl.BlockSpec((1, 1, 128), lambda b, j: (b, 0, 0)),
        out_shape=jax.ShapeDtypeStruct((_NTC, 1, 128), jnp.float32),
    )(x)
    acc = acc[:, 0, :]

    sc = _sc_stats(x).reshape(_NSC, _WPB, 16)

    ymin1 = jnp.concatenate([acc[:, 0], jnp.min(sc[:, :, 0], axis=1)])
    ymax1 = jnp.concatenate([acc[:, 1], jnp.max(sc[:, :, 1], axis=1)])
    ymin2 = jnp.concatenate([acc[:, 2], jnp.min(sc[:, :, 2], axis=1)])
    ymax2 = jnp.concatenate([acc[:, 3], jnp.max(sc[:, :, 3], axis=1)])
    p1sum = jnp.concatenate([acc[:, 4], jnp.sum(sc[:, :, 4], axis=1)])
    p2sum = jnp.concatenate([acc[:, 5], jnp.sum(sc[:, :, 5], axis=1)])

    h1 = jnp.where(ymax1 >= 0.0, ymax1 - ymin1, 0.0)
    h2 = jnp.where(ymax2 >= 0.0, ymax2 - ymin2, 0.0)
    cdr = h1 / (h2 + 1e-06)
    scale = 1.0 / (_H * _W)
    cup_mean = p1sum * scale
    disc_mean = p2sum * scale
    return jnp.stack([cdr, disc_mean, cup_mean, disc_mean, cup_mean], axis=1)


# speedup vs baseline: 1.0777x; 1.0777x over previous
import dataclasses
import functools

import jax
import jax.numpy as jnp
from jax.experimental import pallas as pl
from jax.experimental.pallas import tpu as pltpu
from jax.experimental.pallas import tpu_sc as plsc

_B, _C, _H, _W = 16, 3, 512, 512

_NTC = 8
_DEPTH = 4
_NSC = _B - _NTC
_NCORES, _NSUB, _L = 2, 16, 16
_NW = _NCORES * _NSUB
_WPB = _NW // _NSC
_RW = _H // _WPB
_RC = 16
_CHUNKS = _RW // _RC


def _tc_kernel(x_hbm, acc_ref, bufs, sems):
    def copies(b):
        slot = b % _DEPTH
        return [pltpu.make_async_copy(x_hbm.at[b, c], bufs.at[slot, c],
                                      sems.at[slot])
                for c in range(_C)]

    for b in range(min(_DEPTH, _NTC)):
        for cp_ in copies(b):
            cp_.start()

    for b in range(_NTC):
        for cp_ in copies(b):
            cp_.wait()
        x = bufs[b % _DEPTH]
        c0, c1, c2 = x[0], x[1], x[2]

        d1 = c1 - c0
        d2 = c2 - c0
        r1 = jnp.exp(d1)
        r2 = jnp.exp(d2)
        inv = 1.0 / (1.0 + r1 + r2)
        p1sum = jnp.sum(r1 * inv)
        p2sum = jnp.sum(r2 * inv)

        a1 = (d1 > 0.0) & (d1 >= d2)
        a2 = (d2 > 0.0) & (d2 > d1)
        rowhas1 = jnp.any(a1, axis=1)
        rowhas2 = jnp.any(a2, axis=1)

        rows = jax.lax.iota(jnp.int32, _H).astype(jnp.float32)
        big = jnp.float32(_H)
        ymin1 = jnp.min(jnp.where(rowhas1, rows, big))
        ymax1 = jnp.max(jnp.where(rowhas1, rows, -1.0))
        ymin2 = jnp.min(jnp.where(rowhas2, rows, big))
        ymax2 = jnp.max(jnp.where(rowhas2, rows, -1.0))

        lane = jax.lax.broadcasted_iota(jnp.int32, (1, 128), 1)
        vals = jnp.zeros((1, 128), jnp.float32)
        vals = jnp.where(lane == 0, ymin1, vals)
        vals = jnp.where(lane == 1, ymax1, vals)
        vals = jnp.where(lane == 2, ymin2, vals)
        vals = jnp.where(lane == 3, ymax2, vals)
        vals = jnp.where(lane == 4, p1sum, vals)
        vals = jnp.where(lane == 5, p2sum, vals)
        acc_ref[b] = vals
        nb = b + _DEPTH
        if nb < _NTC:
            for cp_ in copies(nb):
                cp_.start()


def _sc_stats(x):
    mesh = plsc.VectorSubcoreMesh(core_axis_name="c", subcore_axis_name="s")
    cp = pltpu.CompilerParams()
    if "needs_layout_passes" in pltpu.CompilerParams.__dataclass_fields__:
        cp = dataclasses.replace(cp, needs_layout_passes=False)

    @functools.partial(
        pl.kernel, mesh=mesh, compiler_params=cp,
        out_type=jax.ShapeDtypeStruct((_NW, 16), jnp.float32),
        scratch_types=[
            pltpu.VMEM((2, _C, _RC, _W), jnp.float32),
            pltpu.VMEM((16,), jnp.float32),
            pltpu.SemaphoreType.DMA,
            pltpu.SemaphoreType.DMA,
        ],
    )
    def sc_kernel(x_hbm, out_hbm, buf, outv, sem0, sem1):
        wid = jax.lax.axis_index("c") * _NSUB + jax.lax.axis_index("s")
        b = _NTC + wid // _WPB
        row0 = (wid % _WPB) * _RW
        sems = (sem0, sem1)

        def issue(k):
            slot = k % 2
            rs = row0 + k * _RC
            return [pltpu.async_copy(x_hbm.at[b, c, pl.ds(rs, _RC), :],
                                     buf.at[slot, c], sems[slot])
                    for c in range(_C)]

        zero = jnp.zeros((16,), jnp.float32)
        mzero = jnp.zeros((16,), jnp.bool_)
        carry = (zero, zero, jnp.float32(_H), jnp.float32(-1.0),
                 jnp.float32(_H), jnp.float32(-1.0))

        pending = issue(0)
        for k in range(_CHUNKS):
            nxt = issue(k + 1) if k + 1 < _CHUNKS else None
            for cp_ in pending:
                cp_.wait()
            slot = k % 2
            rs = row0 + k * _RC

            def row_body(r, cr, slot=slot, rs=rs):
                p1, p2, y1n, y1x, y2n, y2x = cr

                def w_body(wi, cr2):
                    p1, p2, m1a, m2a = cr2
                    w0 = wi * _L
                    v0 = buf[slot, 0, r, pl.ds(w0, _L)]
                    v1 = buf[slot, 1, r, pl.ds(w0, _L)]
                    v2 = buf[slot, 2, r, pl.ds(w0, _L)]
                    d1 = v1 - v0
                    d2 = v2 - v0
                    e1 = jnp.exp(d1)
                    e2 = jnp.exp(d2)
                    inv = 1.0 / (1.0 + e1 + e2)
                    p1 = p1 + e1 * inv
                    p2 = p2 + e2 * inv
                    m1a = m1a | ((d1 > 0.0) & (d1 >= d2))
                    m2a = m2a | ((d2 > 0.0) & (d2 > d1))
                    return (p1, p2, m1a, m2a)

                p1, p2, m1a, m2a = jax.lax.fori_loop(
                    0, _W // _L, w_body, (p1, p2, mzero, mzero))
                rowf = (rs + r).astype(jnp.float32)
                has1 = jnp.any(m1a)
                has2 = jnp.any(m2a)
                y1n = jnp.where(has1, jnp.minimum(y1n, rowf), y1n)
                y1x = jnp.where(has1, rowf, y1x)
                y2n = jnp.where(has2, jnp.minimum(y2n, rowf), y2n)
                y2x = jnp.where(has2, rowf, y2x)
                return (p1, p2, y1n, y1x, y2n, y2x)

            carry = jax.lax.fori_loop(0, _RC, row_body, carry)
            pending = nxt

        p1, p2, y1n, y1x, y2n, y2x = carry
        lane = jax.lax.iota(jnp.int32, 16)
        res = zero
        res = jnp.where(lane == 0, y1n, res)
        res = jnp.where(lane == 1, y1x, res)
        res = jnp.where(lane == 2, y2n, res)
        res = jnp.where(lane == 3, y2x, res)
        res = jnp.where(lane == 4, jnp.sum(p1), res)
        res = jnp.where(lane == 5, jnp.sum(p2), res)
        outv[...] = res
        pltpu.sync_copy(outv, out_hbm.at[wid])

    return sc_kernel(x)


@jax.jit
def kernel(segmentation_mask):
    x = segmentation_mask

    acc = pl.pallas_call(
        _tc_kernel,
        in_specs=[pl.BlockSpec(memory_space=pl.ANY)],
        out_specs=pl.BlockSpec((_NTC, 1, 128), lambda: (0, 0, 0)),
        out_shape=jax.ShapeDtypeStruct((_NTC, 1, 128), jnp.float32),
        scratch_shapes=[
            pltpu.VMEM((_DEPTH, _C, _H, _W), jnp.float32),
            pltpu.SemaphoreType.DMA((_DEPTH,)),
        ],
    )(x)
    acc = acc[:, 0, :]

    sc = _sc_stats(x).reshape(_NSC, _WPB, 16)

    ymin1 = jnp.concatenate([acc[:, 0], jnp.min(sc[:, :, 0], axis=1)])
    ymax1 = jnp.concatenate([acc[:, 1], jnp.max(sc[:, :, 1], axis=1)])
    ymin2 = jnp.concatenate([acc[:, 2], jnp.min(sc[:, :, 2], axis=1)])
    ymax2 = jnp.concatenate([acc[:, 3], jnp.max(sc[:, :, 3], axis=1)])
    p1sum = jnp.concatenate([acc[:, 4], jnp.sum(sc[:, :, 4], axis=1)])
    p2sum = jnp.concatenate([acc[:, 5], jnp.sum(sc[:, :, 5], axis=1)])

    h1 = jnp.where(ymax1 >= 0.0, ymax1 - ymin1, 0.0)
    h2 = jnp.where(ymax2 >= 0.0, ymax2 - ymin2, 0.0)
    cdr = h1 / (h2 + 1e-06)
    scale = 1.0 / (_H * _W)
    cup_mean = p1sum * scale
    disc_mean = p2sum * scale
    return jnp.stack([cdr, disc_mean, cup_mean, disc_mean, cup_mean], axis=1)


# speedup vs baseline: 1.3371x; 1.2407x over previous
import dataclasses
import functools

import jax
import jax.numpy as jnp
from jax.experimental import pallas as pl
from jax.experimental.pallas import tpu as pltpu
from jax.experimental.pallas import tpu_sc as plsc

_B, _C, _H, _W = 16, 3, 512, 512

_NTC = 12
_DEPTH = 5
_RG = 2
_NSC = _B - _NTC
_NCORES, _NSUB, _L = 2, 16, 16
_NW = _NCORES * _NSUB
_WPB = _NW // _NSC
_RW = _H // _WPB
_RC = 16
_CHUNKS = _RW // _RC


def _tc_kernel(x_hbm, acc_ref, bufs, sems):
    def copies(b):
        slot = b % _DEPTH
        return [pltpu.make_async_copy(x_hbm.at[b, c], bufs.at[slot, c],
                                      sems.at[slot])
                for c in range(_C)]

    for b in range(min(_DEPTH, _NTC)):
        for cp_ in copies(b):
            cp_.start()

    for b in range(_NTC):
        for cp_ in copies(b):
            cp_.wait()
        x = bufs[b % _DEPTH]
        c0, c1, c2 = x[0], x[1], x[2]

        d1 = c1 - c0
        d2 = c2 - c0
        r1 = jnp.exp(d1)
        r2 = jnp.exp(d2)
        inv = 1.0 / (1.0 + r1 + r2)
        p1sum = jnp.sum(r1 * inv)
        p2sum = jnp.sum(r2 * inv)

        a1 = (d1 > 0.0) & (d1 >= d2)
        a2 = (d2 > 0.0) & (d2 > d1)
        rowhas1 = jnp.any(a1, axis=1)
        rowhas2 = jnp.any(a2, axis=1)

        rows = jax.lax.iota(jnp.int32, _H).astype(jnp.float32)
        big = jnp.float32(_H)
        ymin1 = jnp.min(jnp.where(rowhas1, rows, big))
        ymax1 = jnp.max(jnp.where(rowhas1, rows, -1.0))
        ymin2 = jnp.min(jnp.where(rowhas2, rows, big))
        ymax2 = jnp.max(jnp.where(rowhas2, rows, -1.0))

        lane = jax.lax.broadcasted_iota(jnp.int32, (1, 128), 1)
        vals = jnp.zeros((1, 128), jnp.float32)
        vals = jnp.where(lane == 0, ymin1, vals)
        vals = jnp.where(lane == 1, ymax1, vals)
        vals = jnp.where(lane == 2, ymin2, vals)
        vals = jnp.where(lane == 3, ymax2, vals)
        vals = jnp.where(lane == 4, p1sum, vals)
        vals = jnp.where(lane == 5, p2sum, vals)
        acc_ref[b] = vals
        nb = b + _DEPTH
        if nb < _NTC:
            for cp_ in copies(nb):
                cp_.start()


def _sc_stats(x):
    mesh = plsc.VectorSubcoreMesh(core_axis_name="c", subcore_axis_name="s")
    cp = pltpu.CompilerParams()
    if "needs_layout_passes" in pltpu.CompilerParams.__dataclass_fields__:
        cp = dataclasses.replace(cp, needs_layout_passes=False)

    @functools.partial(
        pl.kernel, mesh=mesh, compiler_params=cp,
        out_type=jax.ShapeDtypeStruct((_NW, 16), jnp.float32),
        scratch_types=[
            pltpu.VMEM((2, _C, _RC, _W), jnp.float32),
            pltpu.VMEM((16,), jnp.float32),
            pltpu.SemaphoreType.DMA,
            pltpu.SemaphoreType.DMA,
        ],
    )
    def sc_kernel(x_hbm, out_hbm, buf, outv, sem0, sem1):
        wid = jax.lax.axis_index("c") * _NSUB + jax.lax.axis_index("s")
        b = _NTC + wid // _WPB
        row0 = (wid % _WPB) * _RW
        sems = (sem0, sem1)

        def issue(k):
            slot = k % 2
            rs = row0 + k * _RC
            return [pltpu.async_copy(x_hbm.at[b, c, pl.ds(rs, _RC), :],
                                     buf.at[slot, c], sems[slot])
                    for c in range(_C)]

        zero = jnp.zeros((16,), jnp.float32)
        mzero = jnp.zeros((16,), jnp.bool_)
        carry = (zero, zero, jnp.float32(_H), jnp.float32(-1.0),
                 jnp.float32(_H), jnp.float32(-1.0))

        pending = issue(0)
        for k in range(_CHUNKS):
            nxt = issue(k + 1) if k + 1 < _CHUNKS else None
            for cp_ in pending:
                cp_.wait()
            slot = k % 2
            rs = row0 + k * _RC

            def row_body(r, cr, slot=slot, rs=rs):
                p1, p2, y1n, y1x, y2n, y2x = cr

                def w_body(wi, cr2):
                    p1, p2, m1a, m2a = cr2
                    w0 = wi * _L
                    v0 = buf[slot, 0, r, pl.ds(w0, _L)]
                    v1 = buf[slot, 1, r, pl.ds(w0, _L)]
                    v2 = buf[slot, 2, r, pl.ds(w0, _L)]
                    d1 = v1 - v0
                    d2 = v2 - v0
                    e1 = jnp.exp(d1)
                    e2 = jnp.exp(d2)
                    inv = 1.0 / (1.0 + e1 + e2)
                    p1 = p1 + e1 * inv
                    p2 = p2 + e2 * inv
                    m1a = m1a | ((d1 > 0.0) & (d1 >= d2))
                    m2a = m2a | ((d2 > 0.0) & (d2 > d1))
                    return (p1, p2, m1a, m2a)

                p1, p2, m1a, m2a = jax.lax.fori_loop(
                    0, _W // _L, w_body, (p1, p2, mzero, mzero))
                rowf = (rs + r).astype(jnp.float32)
                has1 = jnp.any(m1a)
                has2 = jnp.any(m2a)
                y1n = jnp.where(has1, jnp.minimum(y1n, rowf), y1n)
                y1x = jnp.where(has1, rowf, y1x)
                y2n = jnp.where(has2, jnp.minimum(y2n, rowf), y2n)
                y2x = jnp.where(has2, rowf, y2x)
                return (p1, p2, y1n, y1x, y2n, y2x)

            carry = jax.lax.fori_loop(0, _RC, row_body, carry)
            pending = nxt

        p1, p2, y1n, y1x, y2n, y2x = carry
        lane = jax.lax.iota(jnp.int32, 16)
        res = zero
        res = jnp.where(lane == 0, y1n, res)
        res = jnp.where(lane == 1, y1x, res)
        res = jnp.where(lane == 2, y2n, res)
        res = jnp.where(lane == 3, y2x, res)
        res = jnp.where(lane == 4, jnp.sum(p1), res)
        res = jnp.where(lane == 5, jnp.sum(p2), res)
        outv[...] = res
        pltpu.sync_copy(outv, out_hbm.at[wid])

    return sc_kernel(x)


@jax.jit
def kernel(segmentation_mask):
    x = segmentation_mask

    acc = pl.pallas_call(
        _tc_kernel,
        in_specs=[pl.BlockSpec(memory_space=pl.ANY)],
        out_specs=pl.BlockSpec((_NTC, 1, 128), lambda: (0, 0, 0)),
        out_shape=jax.ShapeDtypeStruct((_NTC, 1, 128), jnp.float32),
        scratch_shapes=[
            pltpu.VMEM((_DEPTH, _C, _H, _W), jnp.float32),
            pltpu.SemaphoreType.DMA((_DEPTH,)),
        ],
    )(x)
    acc = acc[:, 0, :]

    sc = _sc_stats(x).reshape(_NSC, _WPB, 16)

    ymin1 = jnp.concatenate([acc[:, 0], jnp.min(sc[:, :, 0], axis=1)])
    ymax1 = jnp.concatenate([acc[:, 1], jnp.max(sc[:, :, 1], axis=1)])
    ymin2 = jnp.concatenate([acc[:, 2], jnp.min(sc[:, :, 2], axis=1)])
    ymax2 = jnp.concatenate([acc[:, 3], jnp.max(sc[:, :, 3], axis=1)])
    p1sum = jnp.concatenate([acc[:, 4], jnp.sum(sc[:, :, 4], axis=1)])
    p2sum = jnp.concatenate([acc[:, 5], jnp.sum(sc[:, :, 5], axis=1)])

    h1 = jnp.where(ymax1 >= 0.0, ymax1 - ymin1, 0.0)
    h2 = jnp.where(ymax2 >= 0.0, ymax2 - ymin2, 0.0)
    cdr = h1 / (h2 + 1e-06)
    scale = 1.0 / (_H * _W)
    cup_mean = p1sum * scale
    disc_mean = p2sum * scale
    return jnp.stack([cdr, disc_mean, cup_mean, disc_mean, cup_mean], axis=1)


# speedup vs baseline: 1.3408x; 1.0028x over previous
import dataclasses
import functools

import jax
import jax.numpy as jnp
from jax.experimental import pallas as pl
from jax.experimental.pallas import tpu as pltpu
from jax.experimental.pallas import tpu_sc as plsc

_B, _C, _H, _W = 16, 3, 512, 512

_NTC = 12
_DEPTH = 6
_RG = 2
_NSC = _B - _NTC
_NCORES, _NSUB, _L = 2, 16, 16
_NW = _NCORES * _NSUB
_WPB = _NW // _NSC
_RW = _H // _WPB
_RC = 16
_CHUNKS = _RW // _RC


def _tc_kernel(x_hbm, acc_ref, bufs, sems):
    def copies(b):
        slot = b % _DEPTH
        return [pltpu.make_async_copy(x_hbm.at[b, c], bufs.at[slot, c],
                                      sems.at[slot])
                for c in range(_C)]

    for b in range(min(_DEPTH, _NTC)):
        for cp_ in copies(b):
            cp_.start()

    for b in range(_NTC):
        for cp_ in copies(b):
            cp_.wait()
        x = bufs[b % _DEPTH]
        c0, c1, c2 = x[0], x[1], x[2]

        d1 = c1 - c0
        d2 = c2 - c0
        r1 = jnp.exp(d1)
        r2 = jnp.exp(d2)
        inv = 1.0 / (1.0 + r1 + r2)
        p1sum = jnp.sum(r1 * inv)
        p2sum = jnp.sum(r2 * inv)

        a1 = (d1 > 0.0) & (d1 >= d2)
        a2 = (d2 > 0.0) & (d2 > d1)
        rowhas1 = jnp.any(a1, axis=1)
        rowhas2 = jnp.any(a2, axis=1)

        rows = jax.lax.iota(jnp.int32, _H).astype(jnp.float32)
        big = jnp.float32(_H)
        ymin1 = jnp.min(jnp.where(rowhas1, rows, big))
        ymax1 = jnp.max(jnp.where(rowhas1, rows, -1.0))
        ymin2 = jnp.min(jnp.where(rowhas2, rows, big))
        ymax2 = jnp.max(jnp.where(rowhas2, rows, -1.0))

        lane = jax.lax.broadcasted_iota(jnp.int32, (1, 128), 1)
        vals = jnp.zeros((1, 128), jnp.float32)
        vals = jnp.where(lane == 0, ymin1, vals)
        vals = jnp.where(lane == 1, ymax1, vals)
        vals = jnp.where(lane == 2, ymin2, vals)
        vals = jnp.where(lane == 3, ymax2, vals)
        vals = jnp.where(lane == 4, p1sum, vals)
        vals = jnp.where(lane == 5, p2sum, vals)
        acc_ref[b] = vals
        nb = b + _DEPTH
        if nb < _NTC:
            for cp_ in copies(nb):
                cp_.start()


def _sc_stats(x):
    mesh = plsc.VectorSubcoreMesh(core_axis_name="c", subcore_axis_name="s")
    cp = pltpu.CompilerParams()
    if "needs_layout_passes" in pltpu.CompilerParams.__dataclass_fields__:
        cp = dataclasses.replace(cp, needs_layout_passes=False)

    @functools.partial(
        pl.kernel, mesh=mesh, compiler_params=cp,
        out_type=jax.ShapeDtypeStruct((_NW, 16), jnp.float32),
        scratch_types=[
            pltpu.VMEM((3, _C, _RC, _W), jnp.float32),
            pltpu.VMEM((16,), jnp.float32),
            pltpu.SemaphoreType.DMA,
            pltpu.SemaphoreType.DMA,
            pltpu.SemaphoreType.DMA,
        ],
    )
    def sc_kernel(x_hbm, out_hbm, buf, outv, sem0, sem1, sem2):
        wid = jax.lax.axis_index("c") * _NSUB + jax.lax.axis_index("s")
        b = _NTC + wid // _WPB
        row0 = (wid % _WPB) * _RW
        sems = (sem0, sem1, sem2)

        def issue(k):
            slot = k % 3
            rs = row0 + k * _RC
            return [pltpu.async_copy(x_hbm.at[b, c, pl.ds(rs, _RC), :],
                                     buf.at[slot, c], sems[slot])
                    for c in range(_C)]

        zero = jnp.zeros((16,), jnp.float32)
        mzero = jnp.zeros((16,), jnp.bool_)
        carry = (zero, zero, jnp.float32(_H), jnp.float32(-1.0),
                 jnp.float32(_H), jnp.float32(-1.0))

        queue = [issue(0)]
        if _CHUNKS > 1:
            queue.append(issue(1))
        for k in range(_CHUNKS):
            for cp_ in queue.pop(0):
                cp_.wait()
            slot = k % 3
            rs = row0 + k * _RC

            def row_body(r, cr, slot=slot, rs=rs):
                p1, p2, y1n, y1x, y2n, y2x = cr

                def w_body(wi, cr2):
                    p1, p2, m1a, m2a = cr2
                    w0 = wi * _L
                    v0 = buf[slot, 0, r, pl.ds(w0, _L)]
                    v1 = buf[slot, 1, r, pl.ds(w0, _L)]
                    v2 = buf[slot, 2, r, pl.ds(w0, _L)]
                    d1 = v1 - v0
                    d2 = v2 - v0
                    e1 = jnp.exp(d1)
                    e2 = jnp.exp(d2)
                    inv = 1.0 / (1.0 + e1 + e2)
                    p1 = p1 + e1 * inv
                    p2 = p2 + e2 * inv
                    m1a = m1a | ((d1 > 0.0) & (d1 >= d2))
                    m2a = m2a | ((d2 > 0.0) & (d2 > d1))
                    return (p1, p2, m1a, m2a)

                p1, p2, m1a, m2a = jax.lax.fori_loop(
                    0, _W // _L, w_body, (p1, p2, mzero, mzero))
                rowf = (rs + r).astype(jnp.float32)
                has1 = jnp.any(m1a)
                has2 = jnp.any(m2a)
                y1n = jnp.where(has1, jnp.minimum(y1n, rowf), y1n)
                y1x = jnp.where(has1, rowf, y1x)
                y2n = jnp.where(has2, jnp.minimum(y2n, rowf), y2n)
                y2x = jnp.where(has2, rowf, y2x)
                return (p1, p2, y1n, y1x, y2n, y2x)

            carry = jax.lax.fori_loop(0, _RC, row_body, carry)
            if k + 2 < _CHUNKS:
                queue.append(issue(k + 2))

        p1, p2, y1n, y1x, y2n, y2x = carry
        lane = jax.lax.iota(jnp.int32, 16)
        res = zero
        res = jnp.where(lane == 0, y1n, res)
        res = jnp.where(lane == 1, y1x, res)
        res = jnp.where(lane == 2, y2n, res)
        res = jnp.where(lane == 3, y2x, res)
        res = jnp.where(lane == 4, jnp.sum(p1), res)
        res = jnp.where(lane == 5, jnp.sum(p2), res)
        outv[...] = res
        pltpu.sync_copy(outv, out_hbm.at[wid])

    return sc_kernel(x)


@jax.jit
def kernel(segmentation_mask):
    x = segmentation_mask

    acc = pl.pallas_call(
        _tc_kernel,
        in_specs=[pl.BlockSpec(memory_space=pl.ANY)],
        out_specs=pl.BlockSpec((_NTC, 1, 128), lambda: (0, 0, 0)),
        out_shape=jax.ShapeDtypeStruct((_NTC, 1, 128), jnp.float32),
        scratch_shapes=[
            pltpu.VMEM((_DEPTH, _C, _H, _W), jnp.float32),
            pltpu.SemaphoreType.DMA((_DEPTH,)),
        ],
    )(x)
    acc = acc[:, 0, :]

    sc = _sc_stats(x).reshape(_NSC, _WPB, 16)

    ymin1 = jnp.concatenate([acc[:, 0], jnp.min(sc[:, :, 0], axis=1)])
    ymax1 = jnp.concatenate([acc[:, 1], jnp.max(sc[:, :, 1], axis=1)])
    ymin2 = jnp.concatenate([acc[:, 2], jnp.min(sc[:, :, 2], axis=1)])
    ymax2 = jnp.concatenate([acc[:, 3], jnp.max(sc[:, :, 3], axis=1)])
    p1sum = jnp.concatenate([acc[:, 4], jnp.sum(sc[:, :, 4], axis=1)])
    p2sum = jnp.concatenate([acc[:, 5], jnp.sum(sc[:, :, 5], axis=1)])

    h1 = jnp.where(ymax1 >= 0.0, ymax1 - ymin1, 0.0)
    h2 = jnp.where(ymax2 >= 0.0, ymax2 - ymin2, 0.0)
    cdr = h1 / (h2 + 1e-06)
    scale = 1.0 / (_H * _W)
    cup_mean = p1sum * scale
    disc_mean = p2sum * scale
    return jnp.stack([cdr, disc_mean, cup_mean, disc_mean, cup_mean], axis=1)
